# Initial kernel scaffold; baseline (speedup 1.0000x reference)
import jax
import jax.numpy as jnp
from jax.experimental import pallas as pl


def kernel(hidden_states, W_gate, beta_coeff, top_k):
    raise NotImplementedError("write your pallas kernel here")


# speedup vs baseline: 1.0872x; 1.0872x over previous
import functools

import jax
import jax.numpy as jnp
from jax.experimental import pallas as pl
from jax.experimental.pallas import tpu as pltpu

D_MODEL = 4096
NUM_EXPERTS = 64
N_TOK = 32768
K_MAX = 0.3
LOGIT_STD_EMA = 1.0
LOGIT_STD_ALPHA = 0.99

T1 = 256
T2 = 2048


def _pass1(x_ref, wt_ref, z_ref, stats_ref, acc_ref):
    z = jnp.dot(x_ref[...], wt_ref[...], preferred_element_type=jnp.float32)
    z_ref[...] = z
    i = pl.program_id(0)

    @pl.when(i == 0)
    def _():
        acc_ref[0] = 0.0
        acc_ref[1] = 0.0

    acc_ref[0] += jnp.sum(z)
    acc_ref[1] += jnp.sum(z * z)

    @pl.when(i == pl.num_programs(0) - 1)
    def _():
        stats_ref[0] = acc_ref[0]
        stats_ref[1] = acc_ref[1]


def _pass2(stats_ref, beta_ref, z_ref, zb_ref, p_ref, idx_ref):
    n = float(N_TOK * NUM_EXPERTS)
    s = stats_ref[0]
    ss = stats_ref[1]
    var = (ss - s * s / n) / (n - 1.0)
    ema = LOGIT_STD_ALPHA * LOGIT_STD_EMA + (1.0 - LOGIT_STD_ALPHA) * jnp.sqrt(var)
    beta_eff = jnp.clip(beta_ref[...], -K_MAX, K_MAX) * ema
    zb = z_ref[...] + beta_eff
    zb_ref[...] = zb
    iota = jax.lax.broadcasted_iota(jnp.int32, zb.shape, 1)
    big = jnp.int32(NUM_EXPERTS)
    m1 = jnp.max(zb, axis=1, keepdims=True)
    i1 = jnp.min(jnp.where(zb == m1, iota, big), axis=1, keepdims=True)
    masked = jnp.where(iota == i1, -jnp.inf, zb)
    m2 = jnp.max(masked, axis=1, keepdims=True)
    i2 = jnp.min(jnp.where(masked == m2, iota, big), axis=1, keepdims=True)
    e2 = jnp.exp(m2 - m1)
    denom = 1.0 + e2
    p_ref[...] = jnp.concatenate([1.0 / denom, e2 / denom], axis=1)
    idx_ref[...] = jnp.concatenate([i1, i2], axis=1)


@functools.partial(jax.jit, static_argnames=())
def kernel(hidden_states, W_gate, beta_coeff, top_k):
    del top_k
    wt = W_gate.T
    beta2d = beta_coeff.reshape(1, NUM_EXPERTS)

    z_clean, stats = pl.pallas_call(
        _pass1,
        grid=(N_TOK // T1,),
        in_specs=[
            pl.BlockSpec((T1, D_MODEL), lambda i: (i, 0)),
            pl.BlockSpec((D_MODEL, NUM_EXPERTS), lambda i: (0, 0)),
        ],
        out_specs=[
            pl.BlockSpec((T1, NUM_EXPERTS), lambda i: (i, 0)),
            pl.BlockSpec(memory_space=pltpu.SMEM),
        ],
        out_shape=[
            jax.ShapeDtypeStruct((N_TOK, NUM_EXPERTS), jnp.float32),
            jax.ShapeDtypeStruct((2,), jnp.float32),
        ],
        scratch_shapes=[pltpu.SMEM((2,), jnp.float32)],
    )(hidden_states, wt)

    zb, probs, idx = pl.pallas_call(
        _pass2,
        grid=(N_TOK // T2,),
        in_specs=[
            pl.BlockSpec(memory_space=pltpu.SMEM),
            pl.BlockSpec((1, NUM_EXPERTS), lambda i: (0, 0)),
            pl.BlockSpec((T2, NUM_EXPERTS), lambda i: (i, 0)),
        ],
        out_specs=[
            pl.BlockSpec((T2, NUM_EXPERTS), lambda i: (i, 0)),
            pl.BlockSpec((T2, 2), lambda i: (i, 0)),
            pl.BlockSpec((T2, 2), lambda i: (i, 0)),
        ],
        out_shape=[
            jax.ShapeDtypeStruct((N_TOK, NUM_EXPERTS), jnp.float32),
            jax.ShapeDtypeStruct((N_TOK, 2), jnp.float32),
            jax.ShapeDtypeStruct((N_TOK, 2), jnp.int32),
        ],
    )(stats, beta2d, z_clean)

    return probs, idx, z_clean, zb


# speedup vs baseline: 1.2970x; 1.1930x over previous
import functools

import jax
import jax.numpy as jnp
from jax.experimental import pallas as pl
from jax.experimental.pallas import tpu as pltpu

D_MODEL = 4096
NUM_EXPERTS = 64
N_TOK = 32768
K_MAX = 0.3
LOGIT_STD_EMA = 1.0
LOGIT_STD_ALPHA = 0.99

T1 = 512
T2 = 2048


def _pass1(x_ref, wt_ref, z_ref, stats_ref, acc_ref):
    z = jnp.dot(x_ref[...], wt_ref[...], preferred_element_type=jnp.float32)
    z_ref[...] = z
    i = pl.program_id(0)

    @pl.when(i == 0)
    def _():
        acc_ref[0] = 0.0
        acc_ref[1] = 0.0

    acc_ref[0] += jnp.sum(z)
    acc_ref[1] += jnp.sum(z * z)

    @pl.when(i == pl.num_programs(0) - 1)
    def _():
        stats_ref[0] = acc_ref[0]
        stats_ref[1] = acc_ref[1]


def _pass2(stats_ref, beta_ref, z_ref, zb_ref, p_ref, idx_ref):
    n = float(N_TOK * NUM_EXPERTS)
    s = stats_ref[0]
    ss = stats_ref[1]
    var = (ss - s * s / n) / (n - 1.0)
    ema = LOGIT_STD_ALPHA * LOGIT_STD_EMA + (1.0 - LOGIT_STD_ALPHA) * jnp.sqrt(var)
    beta_eff = jnp.clip(beta_ref[...], -K_MAX, K_MAX) * ema
    zb = z_ref[...] + beta_eff
    zb_ref[...] = zb
    fiota = jax.lax.broadcasted_iota(jnp.int32, zb.shape, 1).astype(jnp.float32)
    big = jnp.float32(NUM_EXPERTS)
    m1 = jnp.max(zb, axis=1, keepdims=True)
    i1 = jnp.min(jnp.where(zb == m1, fiota, big), axis=1, keepdims=True)
    masked = jnp.where(fiota == i1, -jnp.inf, zb)
    m2 = jnp.max(masked, axis=1, keepdims=True)
    i2 = jnp.min(jnp.where(masked == m2, fiota, big), axis=1, keepdims=True)
    e2 = jnp.exp(m2 - m1)
    denom = 1.0 + e2
    p_ref[...] = jnp.concatenate([1.0 / denom, e2 / denom], axis=1)
    idx_ref[...] = jnp.concatenate([i1, i2], axis=1).astype(jnp.int32)


@functools.partial(jax.jit, static_argnames=())
def kernel(hidden_states, W_gate, beta_coeff, top_k):
    del top_k
    wt = W_gate.T
    beta2d = beta_coeff.reshape(1, NUM_EXPERTS)

    z_clean, stats = pl.pallas_call(
        _pass1,
        grid=(N_TOK // T1,),
        in_specs=[
            pl.BlockSpec((T1, D_MODEL), lambda i: (i, 0)),
            pl.BlockSpec((D_MODEL, NUM_EXPERTS), lambda i: (0, 0)),
        ],
        out_specs=[
            pl.BlockSpec((T1, NUM_EXPERTS), lambda i: (i, 0)),
            pl.BlockSpec(memory_space=pltpu.SMEM),
        ],
        out_shape=[
            jax.ShapeDtypeStruct((N_TOK, NUM_EXPERTS), jnp.float32),
            jax.ShapeDtypeStruct((2,), jnp.float32),
        ],
        scratch_shapes=[pltpu.SMEM((2,), jnp.float32)],
    )(hidden_states, wt)

    zb, probs, idx = pl.pallas_call(
        _pass2,
        grid=(N_TOK // T2,),
        in_specs=[
            pl.BlockSpec(memory_space=pltpu.SMEM),
            pl.BlockSpec((1, NUM_EXPERTS), lambda i: (0, 0)),
            pl.BlockSpec((T2, NUM_EXPERTS), lambda i: (i, 0)),
        ],
        out_specs=[
            pl.BlockSpec((T2, NUM_EXPERTS), lambda i: (i, 0)),
            pl.BlockSpec((T2, 2), lambda i: (i, 0)),
            pl.BlockSpec((T2, 2), lambda i: (i, 0)),
        ],
        out_shape=[
            jax.ShapeDtypeStruct((N_TOK, NUM_EXPERTS), jnp.float32),
            jax.ShapeDtypeStruct((N_TOK, 2), jnp.float32),
            jax.ShapeDtypeStruct((N_TOK, 2), jnp.int32),
        ],
    )(stats, beta2d, z_clean)

    return probs, idx, z_clean, zb


# speedup vs baseline: 1.3230x; 1.0201x over previous
import functools

import jax
import jax.numpy as jnp
from jax.experimental import pallas as pl
from jax.experimental.pallas import tpu as pltpu

D_MODEL = 4096
NUM_EXPERTS = 64
N_TOK = 32768
K_MAX = 0.3
LOGIT_STD_EMA = 1.0
LOGIT_STD_ALPHA = 0.99

T1 = 1024
T2 = 4096


def _pass1(x_ref, wt_ref, z_ref, stats_ref, acc_ref):
    z = jnp.dot(x_ref[...], wt_ref[...], preferred_element_type=jnp.float32)
    z_ref[...] = z
    i = pl.program_id(0)

    @pl.when(i == 0)
    def _():
        acc_ref[0] = 0.0
        acc_ref[1] = 0.0

    acc_ref[0] += jnp.sum(z)
    acc_ref[1] += jnp.sum(z * z)

    @pl.when(i == pl.num_programs(0) - 1)
    def _():
        stats_ref[0] = acc_ref[0]
        stats_ref[1] = acc_ref[1]


def _pass2(stats_ref, beta_ref, z_ref, zb_ref, p_ref, idx_ref):
    n = float(N_TOK * NUM_EXPERTS)
    s = stats_ref[0]
    ss = stats_ref[1]
    var = (ss - s * s / n) / (n - 1.0)
    ema = LOGIT_STD_ALPHA * LOGIT_STD_EMA + (1.0 - LOGIT_STD_ALPHA) * jnp.sqrt(var)
    beta_eff = jnp.clip(beta_ref[...], -K_MAX, K_MAX) * ema
    zb = z_ref[...] + beta_eff
    zb_ref[...] = zb
    fiota = jax.lax.broadcasted_iota(jnp.int32, zb.shape, 1).astype(jnp.float32)
    big = jnp.float32(NUM_EXPERTS)
    m1 = jnp.max(zb, axis=1, keepdims=True)
    i1 = jnp.min(jnp.where(zb == m1, fiota, big), axis=1, keepdims=True)
    masked = jnp.where(fiota == i1, -jnp.inf, zb)
    m2 = jnp.max(masked, axis=1, keepdims=True)
    i2 = jnp.min(jnp.where(masked == m2, fiota, big), axis=1, keepdims=True)
    e2 = jnp.exp(m2 - m1)
    denom = 1.0 + e2
    p_ref[...] = jnp.concatenate([1.0 / denom, e2 / denom], axis=1)
    idx_ref[...] = jnp.concatenate([i1, i2], axis=1).astype(jnp.int32)


@functools.partial(jax.jit, static_argnames=())
def kernel(hidden_states, W_gate, beta_coeff, top_k):
    del top_k
    wt = W_gate.T
    beta2d = beta_coeff.reshape(1, NUM_EXPERTS)

    z_clean, stats = pl.pallas_call(
        _pass1,
        grid=(N_TOK // T1,),
        in_specs=[
            pl.BlockSpec((T1, D_MODEL), lambda i: (i, 0)),
            pl.BlockSpec((D_MODEL, NUM_EXPERTS), lambda i: (0, 0)),
        ],
        out_specs=[
            pl.BlockSpec((T1, NUM_EXPERTS), lambda i: (i, 0)),
            pl.BlockSpec(memory_space=pltpu.SMEM),
        ],
        out_shape=[
            jax.ShapeDtypeStruct((N_TOK, NUM_EXPERTS), jnp.float32),
            jax.ShapeDtypeStruct((2,), jnp.float32),
        ],
        scratch_shapes=[pltpu.SMEM((2,), jnp.float32)],
    )(hidden_states, wt)

    zb, probs, idx = pl.pallas_call(
        _pass2,
        grid=(N_TOK // T2,),
        in_specs=[
            pl.BlockSpec(memory_space=pltpu.SMEM),
            pl.BlockSpec((1, NUM_EXPERTS), lambda i: (0, 0)),
            pl.BlockSpec((T2, NUM_EXPERTS), lambda i: (i, 0)),
        ],
        out_specs=[
            pl.BlockSpec((T2, NUM_EXPERTS), lambda i: (i, 0)),
            pl.BlockSpec((T2, 2), lambda i: (i, 0)),
            pl.BlockSpec((T2, 2), lambda i: (i, 0)),
        ],
        out_shape=[
            jax.ShapeDtypeStruct((N_TOK, NUM_EXPERTS), jnp.float32),
            jax.ShapeDtypeStruct((N_TOK, 2), jnp.float32),
            jax.ShapeDtypeStruct((N_TOK, 2), jnp.int32),
        ],
    )(stats, beta2d, z_clean)

    return probs, idx, z_clean, zb
